# baseline (speedup 1.0000x reference)
import functools

import jax
import jax.numpy as jnp
from jax import lax
from jax.experimental import pallas as pl
from jax.experimental.pallas import tpu as pltpu
from jax.experimental.pallas import tpu_sc as plsc

N_NODES = 10000
N_EDGES = 320000
N_TILES = 16
N_CORES = 2
CHUNK = 64
NBUF = 4
SB = 16
NCHUNK = 320
EPAD = N_TILES * NCHUNK * CHUNK
NPAD = 10240
ROWS_PT = NPAD // N_TILES


def _sc_agg():
    mesh = plsc.VectorSubcoreMesh(core_axis_name="c", subcore_axis_name="s")

    @functools.partial(
        pl.kernel,
        out_type=jax.ShapeDtypeStruct((4, NPAD, 128), jnp.float32),
        mesh=mesh,
        scratch_types=[
            pltpu.VMEM((16,), jnp.int32),
            pltpu.VMEM((SB, CHUNK), jnp.int32),
            pltpu.VMEM((SB, CHUNK), jnp.int32),
            pltpu.VMEM((NBUF, CHUNK, 128), jnp.float32),
            pltpu.VMEM_SHARED((NPAD, 128), jnp.float32),
            pltpu.SemaphoreType.DMA,
            pltpu.SemaphoreType.DMA,
        ],
    )
    def kern(h_hbm, src_hbm, dst_hbm, counts_hbm, zeros_hbm, out_hbm,
             cnt_v, src_v, dst_v, rows_v, acc, sem_g, sem_s):
        cid = lax.axis_index("c")
        sid = lax.axis_index("s")
        row0 = sid * ROWS_PT

        pltpu.sync_copy(counts_hbm.at[cid], cnt_v)

        for core in range(N_CORES):
            @pl.when(cid == core)
            def _():
                for bl in range(2):
                    blk = core * 2 + bl
                    h_blk = h_hbm.at[blk]
                    out_blk = out_hbm.at[blk]
                    ridx = core * 2 * N_TILES + 2 * sid + bl
                    n_sb = cnt_v[...][bl]
                    pltpu.sync_copy(zeros_hbm.at[pl.ds(row0, ROWS_PT)],
                                    acc.at[pl.ds(row0, ROWS_PT)])
                    plsc.subcore_barrier()

                    @pl.loop(0, n_sb)
                    def _(s):
                        pltpu.sync_copy(src_hbm.at[ridx, pl.ds(s * SB, SB)],
                                        src_v)
                        pltpu.sync_copy(dst_hbm.at[ridx, pl.ds(s * SB, SB)],
                                        dst_v)
                        g = {}
                        for b in range(NBUF):
                            g[b] = pltpu.async_copy(
                                h_blk.at[src_v.at[b]], rows_v.at[b], sem_g)
                        sd = {}
                        for c in range(SB):
                            b = c % NBUF
                            g[c].wait()
                            sd[c] = pltpu.async_copy(
                                rows_v.at[b], acc.at[dst_v.at[c]], sem_s,
                                add=True)
                            if c + NBUF < SB:
                                sd[c].wait()
                                g[c + NBUF] = pltpu.async_copy(
                                    h_blk.at[src_v.at[c + NBUF]],
                                    rows_v.at[b], sem_g)
                        for c in range(SB - NBUF, SB):
                            sd[c].wait()

                    plsc.subcore_barrier()
                    pltpu.sync_copy(acc.at[pl.ds(row0, ROWS_PT)],
                                    out_blk.at[pl.ds(row0, ROWS_PT)])
                    if bl == 0:
                        plsc.subcore_barrier()

    return kern


def _dot(a, b):
    return lax.dot_general(a, b, (((1,), (0,)), ((), ())),
                           precision=lax.Precision.HIGHEST,
                           preferred_element_type=jnp.float32)


def _mm_root(hb, w, rows=1000):
    nb, n, cb = hb.shape
    k, o = w.shape
    assert k == nb * cb

    def body(h_ref, w_ref, o_ref):
        acc = _dot(h_ref[0], w_ref[pl.ds(0, cb), :])
        for i in range(1, nb):
            acc += _dot(h_ref[i], w_ref[pl.ds(i * cb, cb), :])
        o_ref[...] = acc

    return pl.pallas_call(
        body,
        grid=(n // rows,),
        in_specs=[
            pl.BlockSpec((nb, rows, cb), lambda r: (0, r, 0)),
            pl.BlockSpec((k, o), lambda r: (0, 0)),
        ],
        out_specs=pl.BlockSpec((rows, o), lambda r: (r, 0)),
        out_shape=jax.ShapeDtypeStruct((n, o), jnp.float32),
    )(hb, w)


def _mm_rel(aggb, w, b2d, root, gate, rows=1000):
    nb, npad, cb = aggb.shape
    k, o = w.shape
    n = root.shape[0]
    assert k == nb * cb

    def body(a_ref, w_ref, b_ref, r_ref, g_ref, o_ref):
        acc = _dot(a_ref[0], w_ref[pl.ds(0, cb), :])
        for i in range(1, nb):
            acc += _dot(a_ref[i], w_ref[pl.ds(i * cb, cb), :])
        acc = acc + b_ref[...] + r_ref[...]
        acc = jnp.maximum(acc, acc * g_ref[0])
        for i in range(o // 128):
            o_ref[i] = acc[:, i * 128:(i + 1) * 128]

    return pl.pallas_call(
        body,
        grid=(n // rows,),
        in_specs=[
            pl.BlockSpec((nb, rows, cb), lambda r: (0, r, 0)),
            pl.BlockSpec((k, o), lambda r: (0, 0)),
            pl.BlockSpec((1, o), lambda r: (0, 0)),
            pl.BlockSpec((rows, o), lambda r: (r, 0)),
            pl.BlockSpec(memory_space=pltpu.SMEM),
        ],
        out_specs=pl.BlockSpec((o // 128, rows, 128), lambda r: (0, r, 0)),
        out_shape=jax.ShapeDtypeStruct((o // 128, n, 128), jnp.float32),
    )(aggb, w, b2d, root, gate)


def _pool_mlp(h3b, batch3d, w1, b1, w2, b2, w3, b3, w4, b4, g=64, rows=1000):
    nb, n, cb = h3b.shape
    hdim = nb * cb
    nt = n // rows

    def body(h_ref, bt_ref, w1r, b1r, w2r, b2r, w3r, b3r, w4r, b4r,
             o_ref, acc):
        step = pl.program_id(0)

        @pl.when(step == 0)
        def _():
            acc[...] = jnp.zeros((g, hdim), jnp.float32)

        gids = lax.broadcasted_iota(jnp.int32, (g, rows), 0)
        onehot = jnp.where(gids == bt_ref[0], 1.0, 0.0).astype(jnp.float32)
        for k in range(nb):
            acc[:, k * cb:(k + 1) * cb] += _dot(onehot, h_ref[k])

        @pl.when(step == nt - 1)
        def _():
            h = jnp.maximum(_dot(acc[...], w1r[...]) + b1r[...], 0.0)
            h = jnp.maximum(_dot(h, w2r[...]) + b2r[...], 0.0)
            h = jnp.maximum(_dot(h, w3r[...]) + b3r[...], 0.0)
            o_ref[...] = _dot(h, w4r[...]) + b4r[...]

    def full(a):
        nd = len(a.shape)
        return pl.BlockSpec(a.shape, lambda r, _nd=nd: (0,) * _nd)

    return pl.pallas_call(
        body,
        grid=(nt,),
        in_specs=[
            pl.BlockSpec((nb, rows, cb), lambda r: (0, r, 0)),
            pl.BlockSpec((1, 1, rows), lambda r: (r, 0, 0)),
            full(w1), full(b1), full(w2), full(b2),
            full(w3), full(b3), full(w4), full(b4),
        ],
        out_specs=pl.BlockSpec((g, 1), lambda r: (0, 0)),
        out_shape=jax.ShapeDtypeStruct((g, 1), jnp.float32),
        scratch_shapes=[pltpu.VMEM((g, hdim), jnp.float32)],
    )(h3b, batch3d, w1, b1, w2, b2, w3, b3, w4, b4)


def kernel(x, edge_index, batch,
           Wrel1, brel1, Wroot1,
           Wrel2, brel2, Wroot2,
           Wrel3, brel3, Wroot3,
           W1, b1, W2, b2, W3, b3, W4, b4):
    src = edge_index[0]
    dst = edge_index[1]
    pad = EPAD - N_EDGES
    src_p = jnp.pad(src, (0, pad))
    dst_p = jnp.pad(dst, (0, pad), constant_values=N_NODES)

    def ch_rows(a):
        r = a.reshape(N_TILES, NCHUNK, CHUNK)
        return jnp.broadcast_to(
            r[None, :, None], (2, N_TILES, 2, NCHUNK, CHUNK)
        ).reshape(2 * 2 * N_TILES, NCHUNK, CHUNK)

    def es_rows(a):
        r = a.reshape(2, N_TILES, NCHUNK // 2, CHUNK)
        z = jnp.zeros((2, N_TILES, 2, NCHUNK, CHUNK), jnp.int32)
        return z.at[:, :, 0, :NCHUNK // 2].set(r).reshape(
            2 * 2 * N_TILES, NCHUNK, CHUNK)

    src_c, dst_c = ch_rows(src_p), ch_rows(dst_p)
    src_e, dst_e = es_rows(src_p), es_rows(dst_p)
    ngrp = NCHUNK // SB
    counts_c = jnp.full((2, 16), ngrp, jnp.int32)
    counts_e = jnp.tile(jnp.array([[ngrp // 2] + [0] * 15], jnp.int32), (2, 1))
    zeros128 = jnp.zeros((NPAD, 128), jnp.float32)
    batch3d = batch.reshape(10, 1, 1000)

    agg_f = _sc_agg()

    def r2(v):
        return v.reshape(1, -1)

    z128 = jnp.zeros((128, 512), jnp.float32)
    z384 = jnp.zeros((384, 512), jnp.float32)
    Wrel_s = jnp.stack([
        jnp.concatenate([Wrel1, z128, Wrel1, z128], axis=0),
        Wrel2, Wrel3])
    Wroot_s = jnp.stack([
        jnp.concatenate([Wroot1, z384], axis=0),
        Wroot2, Wroot3])
    brel_s = jnp.stack([r2(brel1), r2(brel2), r2(brel3)])
    src_s = jnp.stack([src_e, src_c, src_c])
    dst_s = jnp.stack([dst_e, dst_c, dst_c])
    counts_s = jnp.stack([counts_e, counts_c, counts_c])
    gate_s = jnp.array([[0.0], [0.0], [1.0]], jnp.float32)

    xb4 = jnp.tile(x.reshape(1, N_NODES, 128), (4, 1, 1))

    def layer(h, xs):
        src_r, dst_r, counts, wrel, brel2d, wroot, gate = xs
        root = _mm_root(h, wroot)
        agg = agg_f(h, src_r, dst_r, counts, zeros128)
        h_next = _mm_rel(agg, wrel, brel2d, root, gate)
        return h_next, None

    h3b, _ = lax.scan(
        layer, xb4,
        (src_s, dst_s, counts_s, Wrel_s, brel_s, Wroot_s, gate_s),
        unroll=1)

    return _pool_mlp(h3b, batch3d, W1, r2(b1), W2, r2(b2), W3, r2(b3),
                     W4, r2(b4))


# speedup vs baseline: 1.0881x; 1.0881x over previous
import functools

import jax
import jax.numpy as jnp
from jax import lax
from jax.experimental import pallas as pl
from jax.experimental.pallas import tpu as pltpu
from jax.experimental.pallas import tpu_sc as plsc

N_NODES = 10000
N_EDGES = 320000
N_TILES = 16
N_CORES = 2
CHUNK = 128
NBUF = 2
SB = 16
NCHUNK = 160
EPAD = N_TILES * NCHUNK * CHUNK
NPAD = 10240
ROWS_PT = NPAD // N_TILES


def _sc_agg():
    mesh = plsc.VectorSubcoreMesh(core_axis_name="c", subcore_axis_name="s")

    @functools.partial(
        pl.kernel,
        out_type=jax.ShapeDtypeStruct((4, NPAD, 128), jnp.float32),
        mesh=mesh,
        scratch_types=[
            pltpu.VMEM((16,), jnp.int32),
            pltpu.VMEM((SB, CHUNK), jnp.int32),
            pltpu.VMEM((SB, CHUNK), jnp.int32),
            pltpu.VMEM((NBUF, CHUNK, 128), jnp.float32),
            pltpu.VMEM_SHARED((NPAD, 128), jnp.float32),
            pltpu.SemaphoreType.DMA,
            pltpu.SemaphoreType.DMA,
        ],
    )
    def kern(h_hbm, src_hbm, dst_hbm, counts_hbm, zeros_hbm, out_hbm,
             cnt_v, src_v, dst_v, rows_v, acc, sem_g, sem_s):
        cid = lax.axis_index("c")
        sid = lax.axis_index("s")
        row0 = sid * ROWS_PT

        pltpu.sync_copy(counts_hbm.at[cid], cnt_v)

        for core in range(N_CORES):
            @pl.when(cid == core)
            def _():
                for bl in range(2):
                    blk = core * 2 + bl
                    h_blk = h_hbm.at[blk]
                    out_blk = out_hbm.at[blk]
                    ridx = core * 2 * N_TILES + 2 * sid + bl
                    n_sb = cnt_v[...][bl]
                    pltpu.sync_copy(zeros_hbm.at[pl.ds(row0, ROWS_PT)],
                                    acc.at[pl.ds(row0, ROWS_PT)])
                    plsc.subcore_barrier()

                    @pl.loop(0, n_sb)
                    def _(s):
                        pltpu.sync_copy(src_hbm.at[ridx, pl.ds(s * SB, SB)],
                                        src_v)
                        pltpu.sync_copy(dst_hbm.at[ridx, pl.ds(s * SB, SB)],
                                        dst_v)
                        g = {}
                        for b in range(NBUF):
                            g[b] = pltpu.async_copy(
                                h_blk.at[src_v.at[b]], rows_v.at[b], sem_g)
                        sd = {}
                        for c in range(SB):
                            b = c % NBUF
                            g[c].wait()
                            sd[c] = pltpu.async_copy(
                                rows_v.at[b], acc.at[dst_v.at[c]], sem_s,
                                add=True)
                            if c + NBUF < SB:
                                sd[c].wait()
                                g[c + NBUF] = pltpu.async_copy(
                                    h_blk.at[src_v.at[c + NBUF]],
                                    rows_v.at[b], sem_g)
                        for c in range(SB - NBUF, SB):
                            sd[c].wait()

                    plsc.subcore_barrier()
                    pltpu.sync_copy(acc.at[pl.ds(row0, ROWS_PT)],
                                    out_blk.at[pl.ds(row0, ROWS_PT)])
                    if bl == 0:
                        plsc.subcore_barrier()

    return kern


def _dot(a, b):
    return lax.dot_general(a, b, (((1,), (0,)), ((), ())),
                           precision=lax.Precision.HIGHEST,
                           preferred_element_type=jnp.float32)


def _mm_root(hb, w, rows=1000):
    nb, n, cb = hb.shape
    k, o = w.shape
    assert k == nb * cb

    def body(h_ref, w_ref, o_ref):
        acc = _dot(h_ref[0], w_ref[pl.ds(0, cb), :])
        for i in range(1, nb):
            acc += _dot(h_ref[i], w_ref[pl.ds(i * cb, cb), :])
        o_ref[...] = acc

    return pl.pallas_call(
        body,
        grid=(n // rows,),
        in_specs=[
            pl.BlockSpec((nb, rows, cb), lambda r: (0, r, 0)),
            pl.BlockSpec((k, o), lambda r: (0, 0)),
        ],
        out_specs=pl.BlockSpec((rows, o), lambda r: (r, 0)),
        out_shape=jax.ShapeDtypeStruct((n, o), jnp.float32),
    )(hb, w)


def _mm_rel(aggb, w, b2d, root, gate, rows=1000):
    nb, npad, cb = aggb.shape
    k, o = w.shape
    n = root.shape[0]
    assert k == nb * cb

    def body(a_ref, w_ref, b_ref, r_ref, g_ref, o_ref):
        acc = _dot(a_ref[0], w_ref[pl.ds(0, cb), :])
        for i in range(1, nb):
            acc += _dot(a_ref[i], w_ref[pl.ds(i * cb, cb), :])
        acc = acc + b_ref[...] + r_ref[...]
        acc = jnp.maximum(acc, acc * g_ref[0])
        for i in range(o // 128):
            o_ref[i] = acc[:, i * 128:(i + 1) * 128]

    return pl.pallas_call(
        body,
        grid=(n // rows,),
        in_specs=[
            pl.BlockSpec((nb, rows, cb), lambda r: (0, r, 0)),
            pl.BlockSpec((k, o), lambda r: (0, 0)),
            pl.BlockSpec((1, o), lambda r: (0, 0)),
            pl.BlockSpec((rows, o), lambda r: (r, 0)),
            pl.BlockSpec(memory_space=pltpu.SMEM),
        ],
        out_specs=pl.BlockSpec((o // 128, rows, 128), lambda r: (0, r, 0)),
        out_shape=jax.ShapeDtypeStruct((o // 128, n, 128), jnp.float32),
    )(aggb, w, b2d, root, gate)


def _pool_mlp(h3b, batch3d, w1, b1, w2, b2, w3, b3, w4, b4, g=64, rows=1000):
    nb, n, cb = h3b.shape
    hdim = nb * cb
    nt = n // rows

    def body(h_ref, bt_ref, w1r, b1r, w2r, b2r, w3r, b3r, w4r, b4r,
             o_ref, acc):
        step = pl.program_id(0)

        @pl.when(step == 0)
        def _():
            acc[...] = jnp.zeros((g, hdim), jnp.float32)

        gids = lax.broadcasted_iota(jnp.int32, (g, rows), 0)
        onehot = jnp.where(gids == bt_ref[0], 1.0, 0.0).astype(jnp.float32)
        for k in range(nb):
            acc[:, k * cb:(k + 1) * cb] += _dot(onehot, h_ref[k])

        @pl.when(step == nt - 1)
        def _():
            h = jnp.maximum(_dot(acc[...], w1r[...]) + b1r[...], 0.0)
            h = jnp.maximum(_dot(h, w2r[...]) + b2r[...], 0.0)
            h = jnp.maximum(_dot(h, w3r[...]) + b3r[...], 0.0)
            o_ref[...] = _dot(h, w4r[...]) + b4r[...]

    def full(a):
        nd = len(a.shape)
        return pl.BlockSpec(a.shape, lambda r, _nd=nd: (0,) * _nd)

    return pl.pallas_call(
        body,
        grid=(nt,),
        in_specs=[
            pl.BlockSpec((nb, rows, cb), lambda r: (0, r, 0)),
            pl.BlockSpec((1, 1, rows), lambda r: (r, 0, 0)),
            full(w1), full(b1), full(w2), full(b2),
            full(w3), full(b3), full(w4), full(b4),
        ],
        out_specs=pl.BlockSpec((g, 1), lambda r: (0, 0)),
        out_shape=jax.ShapeDtypeStruct((g, 1), jnp.float32),
        scratch_shapes=[pltpu.VMEM((g, hdim), jnp.float32)],
    )(h3b, batch3d, w1, b1, w2, b2, w3, b3, w4, b4)


def kernel(x, edge_index, batch,
           Wrel1, brel1, Wroot1,
           Wrel2, brel2, Wroot2,
           Wrel3, brel3, Wroot3,
           W1, b1, W2, b2, W3, b3, W4, b4):
    src = edge_index[0]
    dst = edge_index[1]
    pad = EPAD - N_EDGES
    src_p = jnp.pad(src, (0, pad))
    dst_p = jnp.pad(dst, (0, pad), constant_values=N_NODES)

    def ch_rows(a):
        r = a.reshape(N_TILES, NCHUNK, CHUNK)
        return jnp.broadcast_to(
            r[None, :, None], (2, N_TILES, 2, NCHUNK, CHUNK)
        ).reshape(2 * 2 * N_TILES, NCHUNK, CHUNK)

    def es_rows(a):
        r = a.reshape(2, N_TILES, NCHUNK // 2, CHUNK)
        z = jnp.zeros((2, N_TILES, 2, NCHUNK, CHUNK), jnp.int32)
        return z.at[:, :, 0, :NCHUNK // 2].set(r).reshape(
            2 * 2 * N_TILES, NCHUNK, CHUNK)

    src_c, dst_c = ch_rows(src_p), ch_rows(dst_p)
    src_e, dst_e = es_rows(src_p), es_rows(dst_p)
    ngrp = NCHUNK // SB
    counts_c = jnp.full((2, 16), ngrp, jnp.int32)
    counts_e = jnp.tile(jnp.array([[ngrp // 2] + [0] * 15], jnp.int32), (2, 1))
    zeros128 = jnp.zeros((NPAD, 128), jnp.float32)
    batch3d = batch.reshape(10, 1, 1000)

    agg_f = _sc_agg()

    def r2(v):
        return v.reshape(1, -1)

    z128 = jnp.zeros((128, 512), jnp.float32)
    z384 = jnp.zeros((384, 512), jnp.float32)
    Wrel_s = jnp.stack([
        jnp.concatenate([Wrel1, z128, Wrel1, z128], axis=0),
        Wrel2, Wrel3])
    Wroot_s = jnp.stack([
        jnp.concatenate([Wroot1, z384], axis=0),
        Wroot2, Wroot3])
    brel_s = jnp.stack([r2(brel1), r2(brel2), r2(brel3)])
    src_s = jnp.stack([src_e, src_c, src_c])
    dst_s = jnp.stack([dst_e, dst_c, dst_c])
    counts_s = jnp.stack([counts_e, counts_c, counts_c])
    gate_s = jnp.array([[0.0], [0.0], [1.0]], jnp.float32)

    xb4 = jnp.tile(x.reshape(1, N_NODES, 128), (4, 1, 1))

    def layer(h, xs):
        src_r, dst_r, counts, wrel, brel2d, wroot, gate = xs
        root = _mm_root(h, wroot)
        agg = agg_f(h, src_r, dst_r, counts, zeros128)
        h_next = _mm_rel(agg, wrel, brel2d, root, gate)
        return h_next, None

    h3b, _ = lax.scan(
        layer, xb4,
        (src_s, dst_s, counts_s, Wrel_s, brel_s, Wroot_s, gate_s),
        unroll=1)

    return _pool_mlp(h3b, batch3d, W1, r2(b1), W2, r2(b2), W3, r2(b3),
                     W4, r2(b4))
